# Initial kernel scaffold; baseline (speedup 1.0000x reference)
import jax
import jax.numpy as jnp
from jax.experimental import pallas as pl


def kernel(inputs):
    raise NotImplementedError("write your pallas kernel here")


# speedup vs baseline: 3.1709x; 3.1709x over previous
import functools

import numpy as np
import jax
import jax.numpy as jnp
from jax import lax
from jax.experimental import pallas as pl
from jax.experimental.pallas import tpu as pltpu
from jax.experimental.pallas import tpu_sc as plsc

_NU = 768
_SCALE = float(np.sqrt(float(_NU)))
_HALFPI = float(np.pi / 2.0)
_TROWS = 512


def _inv_timescales() -> jax.Array:
    i = np.arange(_NU, dtype=np.float64)
    inv = 1.0 / np.power(10000.0, 2.0 * i / _NU)
    return jnp.asarray(inv.reshape(1, _NU), dtype=jnp.float32)


def _table_body(inv_ref, out_ref):
    g = pl.program_id(0)
    row = lax.broadcasted_iota(jnp.int32, (_TROWS, _NU), 0) + g * _TROWS
    pos = row.astype(jnp.float32)
    col = lax.broadcasted_iota(jnp.int32, (_TROWS, _NU), 1)
    angle = pos * inv_ref[...]
    angle = angle + jnp.where((col & 1) == 1, jnp.float32(_HALFPI),
                              jnp.float32(0.0))
    val = jnp.sin(angle)
    out_ref[...] = jnp.where(pos == 0.0, jnp.float32(0.0), val) * _SCALE


def _make_table(T: int) -> jax.Array:
    return pl.pallas_call(
        _table_body,
        grid=(T // _TROWS,),
        in_specs=[pl.BlockSpec((1, _NU), lambda i: (0, 0))],
        out_specs=pl.BlockSpec((_TROWS, _NU), lambda i: (i, 0)),
        out_shape=jax.ShapeDtypeStruct((T, _NU), jnp.float32),
    )(_inv_timescales())


@functools.cache
def _make_broadcast(N: int, T: int):
    info = plsc.get_sparse_core_info()
    nw = info.num_cores * info.num_subcores
    rpw = T // nw
    mesh = plsc.VectorSubcoreMesh(core_axis_name="c", subcore_axis_name="s")

    @functools.partial(
        pl.kernel,
        mesh=mesh,
        out_type=jax.ShapeDtypeStruct((N, T, _NU), jnp.float32),
        scratch_types=[
            pltpu.VMEM((rpw, _NU), jnp.float32),
            pltpu.SemaphoreType.DMA,
        ],
    )
    def bcast(table_hbm, out_hbm, buf, sem):
        wid = lax.axis_index("s") * info.num_cores + lax.axis_index("c")
        base = wid * rpw
        pltpu.sync_copy(table_hbm.at[pl.ds(base, rpw)], buf)
        handles = [
            pltpu.async_copy(buf, out_hbm.at[n, pl.ds(base, rpw)], sem)
            for n in range(N)
        ]
        for h in handles:
            h.wait()

    return bcast


def kernel(inputs):
    N, T = inputs.shape[0], inputs.shape[1]
    table = _make_table(T)
    return _make_broadcast(N, T)(table)


# speedup vs baseline: 4.4764x; 1.4117x over previous
import functools

import numpy as np
import jax
import jax.numpy as jnp
from jax import lax
from jax.experimental import pallas as pl
from jax.experimental.pallas import tpu as pltpu
from jax.experimental.pallas import tpu_sc as plsc

_NU = 768
_SCALE = float(np.sqrt(float(_NU)))
_HALFPI = float(np.pi / 2.0)
_TROWS = 128


@functools.cache
def _table_consts(T: int):
    i = np.arange(_NU, dtype=np.float64)
    inv = 1.0 / np.power(10000.0, 2.0 * i / _NU)
    k = np.arange(T // _TROWS, dtype=np.float64)[:, None] * _TROWS
    off = k * inv[None, :]
    nb = T // _TROWS
    return (jnp.asarray(inv.reshape(1, _NU), dtype=jnp.float32),
            jnp.asarray(np.cos(off).reshape(nb, 1, _NU), dtype=jnp.float32),
            jnp.asarray(np.sin(off).reshape(nb, 1, _NU), dtype=jnp.float32))


def _table_body(inv_ref, cos_ref, sin_ref, out_ref, a_ref, b_ref):
    g = pl.program_id(0)

    @pl.when(g == 0)
    def _base():
        row = lax.broadcasted_iota(jnp.int32, (_TROWS, _NU), 0)
        pos = row.astype(jnp.float32)
        col = lax.broadcasted_iota(jnp.int32, (_TROWS, _NU), 1)
        even = (col & 1) == 0
        angle = pos * inv_ref[...]
        s = jnp.sin(angle)
        c = jnp.sin(angle + jnp.float32(_HALFPI))
        a = jnp.where(even, s, c)
        a_ref[...] = a
        b_ref[...] = jnp.where(even, c, -s)
        out_ref[...] = jnp.where(row == 0, jnp.float32(0.0), a) * _SCALE

    @pl.when(g != 0)
    def _rotate():
        out_ref[...] = (a_ref[...] * cos_ref[0] +
                        b_ref[...] * sin_ref[0]) * _SCALE


def _make_table(T: int) -> jax.Array:
    inv, cos_off, sin_off = _table_consts(T)
    return pl.pallas_call(
        _table_body,
        grid=(T // _TROWS,),
        in_specs=[
            pl.BlockSpec((1, _NU), lambda i: (0, 0)),
            pl.BlockSpec((1, 1, _NU), lambda i: (i, 0, 0)),
            pl.BlockSpec((1, 1, _NU), lambda i: (i, 0, 0)),
        ],
        out_specs=pl.BlockSpec((_TROWS, _NU), lambda i: (i, 0)),
        out_shape=jax.ShapeDtypeStruct((T, _NU), jnp.float32),
        scratch_shapes=[
            pltpu.VMEM((_TROWS, _NU), jnp.float32),
            pltpu.VMEM((_TROWS, _NU), jnp.float32),
        ],
    )(inv, cos_off, sin_off)


@functools.cache
def _make_broadcast(N: int, T: int):
    info = plsc.get_sparse_core_info()
    nw = info.num_cores * info.num_subcores
    rpw = T // nw
    mesh = plsc.VectorSubcoreMesh(core_axis_name="c", subcore_axis_name="s")

    @functools.partial(
        pl.kernel,
        mesh=mesh,
        out_type=jax.ShapeDtypeStruct((N, T, _NU), jnp.float32),
        scratch_types=[
            pltpu.VMEM((rpw, _NU), jnp.float32),
            pltpu.SemaphoreType.DMA,
        ],
    )
    def bcast(table_hbm, out_hbm, buf, sem):
        wid = lax.axis_index("s") * info.num_cores + lax.axis_index("c")
        base = wid * rpw
        pltpu.sync_copy(table_hbm.at[pl.ds(base, rpw)], buf)
        handles = [
            pltpu.async_copy(buf, out_hbm.at[n, pl.ds(base, rpw)], sem)
            for n in range(N)
        ]
        for h in handles:
            h.wait()

    return bcast


def kernel(inputs):
    N, T = inputs.shape[0], inputs.shape[1]
    table = _make_table(T)
    return _make_broadcast(N, T)(table)
